# baseline (speedup 1.0000x reference)
import functools

import jax
import jax.numpy as jnp
from jax import lax
from jax.experimental import pallas as pl
from jax.experimental.pallas import tpu as pltpu
from jax.experimental.pallas import tpu_sc as plsc

_NC, _NS = 2, 16
_NW = _NC * _NS
_C = 6144

_BLOCK = 32768

_RGAMMA_COEF = (
    0.14753032712973058,
    0.37403431078058,
    1.7392577756303944,
    -1.8825767707403378,
    0.7507072601362749,
    -0.13718218631689882,
    0.007790298096042144,
    0.0009124720760137377,
    -0.00011280308480421503,
)


def _rgamma(x):
    acc = jnp.float32(_RGAMMA_COEF[-1])
    for c in reversed(_RGAMMA_COEF[:-1]):
        acc = acc * x + jnp.float32(c)
    return acc


def _sc_gather(param_shape, param_shift, param_rate, param_max, ids, n_sc):
    mesh = plsc.VectorSubcoreMesh(core_axis_name="c", subcore_axis_name="s")
    n_var = param_shape.shape[0]
    n_chunks = n_sc // _C
    out_t = [jax.ShapeDtypeStruct((n_sc,), jnp.float32)] * 4
    scratch = ([pltpu.VMEM((n_var, _C), jnp.float32)]
               + [pltpu.VMEM((_C,), jnp.int32)]
               + [pltpu.VMEM((_C,), jnp.float32)] * 4
               + [pltpu.SemaphoreType.DMA])

    @functools.partial(pl.kernel, out_type=out_t, mesh=mesh,
                       scratch_types=scratch)
    def gather_k(shape_h, shift_h, rate_h, max_h, ids_h,
                 o0, o1, o2, o3, tb, idb, r0, r1, r2, r3, sem):
        wid = lax.axis_index("s") * _NC + lax.axis_index("c")
        tabs = (shape_h, shift_h, rate_h, max_h)
        rows = (r0, r1, r2, r3)
        outs = (o0, o1, o2, o3)

        def chunk(i, carry):
            g = i * _NW + wid

            @pl.when(g < n_chunks)
            def _():
                off = g * _C
                pltpu.sync_copy(ids_h.at[pl.ds(off, _C)], idb)
                for t, r in zip(tabs, rows):
                    pltpu.async_copy(t.at[:, pl.ds(off, _C)], tb, sem).wait()

                    def grp(k, carry2):
                        base16 = k * 16
                        ids16 = idb[pl.ds(base16, 16)]
                        acc = tb[0, pl.ds(base16, 16)]
                        for v in range(1, n_var):
                            acc = jnp.where(ids16 == v,
                                            tb[v, pl.ds(base16, 16)], acc)
                        r[pl.ds(base16, 16)] = acc
                        return carry2

                    lax.fori_loop(0, _C // 16, grp, 0)
                for o, r in zip(outs, rows):
                    pltpu.sync_copy(r, o.at[pl.ds(off, _C)])

            return carry

        lax.fori_loop(0, pl.cdiv(n_chunks, _NW), chunk, 0)

    return gather_k(param_shape, param_shift, param_rate, param_max, ids)


def _compute(tnow, time, inf, shape, shift, rate, max_inf):
    t = tnow - time
    d = t - shift
    u = d * rate
    sign = jnp.where(d + 1e-10 > 0.0, 1.0, 0.0)
    val = jnp.exp((shape - 1.0) * jnp.log(u) - u) * _rgamma(shape)
    return max_inf * sign * rate * val * inf


def _math_body(tnow_ref, time_ref, inf_ref, shape_ref, shift_ref,
               rate_ref, max_ref, out_ref):
    out_ref[...] = _compute(
        tnow_ref[0], time_ref[...], inf_ref[...], shape_ref[...],
        shift_ref[...], rate_ref[...], max_ref[...])


def _tail_body(tnow_ref, time_ref, id_ref, inf_ref, shape_ref, shift_ref,
               rate_ref, max_ref, out_ref):
    ids = id_ref[...]
    n_var, blk = shape_ref.shape
    mask = jax.lax.broadcasted_iota(jnp.int32, (n_var, blk), 0) == ids[None, :]

    def sel(ref):
        return jnp.sum(jnp.where(mask, ref[...], 0.0), axis=0)

    out_ref[...] = _compute(
        tnow_ref[0], time_ref[...], inf_ref[...], sel(shape_ref),
        sel(shift_ref), sel(rate_ref), sel(max_ref))


def kernel(infection_time, infection_id, is_infected, param_shape,
           param_shift, param_rate, param_max, timer_now):
    n = infection_time.shape[0]
    n_sc = (n // _C) * _C
    tnow = jnp.asarray(timer_now, jnp.float32).reshape(1)

    gshape, gshift, grate, gmax = _sc_gather(
        param_shape, param_shift, param_rate, param_max, infection_id, n_sc)

    blk = _BLOCK
    vec_spec = pl.BlockSpec((blk,), lambda i: (i,))
    main = pl.pallas_call(
        _math_body,
        grid=(pl.cdiv(n_sc, blk),),
        in_specs=[pl.BlockSpec(memory_space=pltpu.SMEM)] + [vec_spec] * 6,
        out_specs=vec_spec,
        out_shape=jax.ShapeDtypeStruct((n_sc,), jnp.float32),
    )(tnow, infection_time, is_infected, gshape, gshift, grate, gmax)

    if n_sc == n:
        return main

    tail = pl.pallas_call(
        _tail_body,
        out_shape=jax.ShapeDtypeStruct((n - n_sc,), jnp.float32),
        in_specs=[pl.BlockSpec(memory_space=pltpu.SMEM)] + [pl.BlockSpec()] * 7,
    )(tnow, infection_time[n_sc:], infection_id[n_sc:], is_infected[n_sc:],
      param_shape[:, n_sc:], param_shift[:, n_sc:], param_rate[:, n_sc:],
      param_max[:, n_sc:])
    return jnp.concatenate([main, tail])


# speedup vs baseline: 1.1104x; 1.1104x over previous
import functools

import jax
import jax.numpy as jnp
from jax import lax
from jax.experimental import pallas as pl
from jax.experimental.pallas import tpu as pltpu
from jax.experimental.pallas import tpu_sc as plsc

_NC, _NS = 2, 16
_NW = _NC * _NS
_C = 4608

_BLOCK = 32768

_RGAMMA_COEF = (
    0.14753032712973058,
    0.37403431078058,
    1.7392577756303944,
    -1.8825767707403378,
    0.7507072601362749,
    -0.13718218631689882,
    0.007790298096042144,
    0.0009124720760137377,
    -0.00011280308480421503,
)


def _rgamma(x):
    acc = jnp.float32(_RGAMMA_COEF[-1])
    for c in reversed(_RGAMMA_COEF[:-1]):
        acc = acc * x + jnp.float32(c)
    return acc


def _sc_gather(param_shape, param_shift, param_rate, param_max, ids, n_sc):
    mesh = plsc.VectorSubcoreMesh(core_axis_name="c", subcore_axis_name="s")
    n_var = param_shape.shape[0]
    n_chunks = n_sc // _C
    out_t = [jax.ShapeDtypeStruct((n_sc,), jnp.float32)] * 4
    scratch = ([pltpu.VMEM((n_var, _C), jnp.float32)]
               + [pltpu.VMEM((_C,), jnp.int32)]
               + [pltpu.VMEM((_C,), jnp.float32)] * 4
               + [pltpu.SemaphoreType.DMA])

    @functools.partial(pl.kernel, out_type=out_t, mesh=mesh,
                       scratch_types=scratch)
    def gather_k(shape_h, shift_h, rate_h, max_h, ids_h,
                 o0, o1, o2, o3, tb, idb, r0, r1, r2, r3, sem):
        wid = lax.axis_index("s") * _NC + lax.axis_index("c")
        tabs = (shape_h, shift_h, rate_h, max_h)
        rows = (r0, r1, r2, r3)
        outs = (o0, o1, o2, o3)

        def chunk(i, carry):
            g = i * _NW + wid

            @pl.when(g < n_chunks)
            def _():
                off = g * _C
                id_cp = pltpu.async_copy(ids_h.at[pl.ds(off, _C)], idb, sem)
                tab_cp = pltpu.async_copy(
                    tabs[0].at[:, pl.ds(off, _C)], tb, sem)
                id_cp.wait()
                for t_i, r in enumerate(rows):
                    tab_cp.wait()

                    def grp(k, carry2):
                        base16 = k * 16
                        ids16 = idb[pl.ds(base16, 16)]
                        acc = tb[0, pl.ds(base16, 16)]
                        for v in range(1, n_var):
                            acc = jnp.where(ids16 == v,
                                            tb[v, pl.ds(base16, 16)], acc)
                        r[pl.ds(base16, 16)] = acc
                        return carry2

                    lax.fori_loop(0, _C // 16, grp, 0)
                    if t_i < 3:
                        tab_cp = pltpu.async_copy(
                            tabs[t_i + 1].at[:, pl.ds(off, _C)], tb, sem)
                out_cps = [pltpu.async_copy(r, o.at[pl.ds(off, _C)], sem)
                           for o, r in zip(outs, rows)]
                for c in out_cps:
                    c.wait()

            return carry

        lax.fori_loop(0, pl.cdiv(n_chunks, _NW), chunk, 0)

    return gather_k(param_shape, param_shift, param_rate, param_max, ids)


def _compute(tnow, time, inf, shape, shift, rate, max_inf):
    t = tnow - time
    d = t - shift
    u = d * rate
    sign = jnp.where(d + 1e-10 > 0.0, 1.0, 0.0)
    val = jnp.exp((shape - 1.0) * jnp.log(u) - u) * _rgamma(shape)
    return max_inf * sign * rate * val * inf


def _math_body(tnow_ref, time_ref, inf_ref, shape_ref, shift_ref,
               rate_ref, max_ref, out_ref):
    out_ref[...] = _compute(
        tnow_ref[0], time_ref[...], inf_ref[...], shape_ref[...],
        shift_ref[...], rate_ref[...], max_ref[...])


def _tail_body(tnow_ref, time_ref, id_ref, inf_ref, shape_ref, shift_ref,
               rate_ref, max_ref, out_ref):
    ids = id_ref[...]
    n_var, blk = shape_ref.shape
    mask = jax.lax.broadcasted_iota(jnp.int32, (n_var, blk), 0) == ids[None, :]

    def sel(ref):
        return jnp.sum(jnp.where(mask, ref[...], 0.0), axis=0)

    out_ref[...] = _compute(
        tnow_ref[0], time_ref[...], inf_ref[...], sel(shape_ref),
        sel(shift_ref), sel(rate_ref), sel(max_ref))


def kernel(infection_time, infection_id, is_infected, param_shape,
           param_shift, param_rate, param_max, timer_now):
    n = infection_time.shape[0]
    n_sc = (n // _C) * _C
    tnow = jnp.asarray(timer_now, jnp.float32).reshape(1)

    gshape, gshift, grate, gmax = _sc_gather(
        param_shape, param_shift, param_rate, param_max, infection_id, n_sc)

    blk = _BLOCK
    vec_spec = pl.BlockSpec((blk,), lambda i: (i,))
    main = pl.pallas_call(
        _math_body,
        grid=(pl.cdiv(n_sc, blk),),
        in_specs=[pl.BlockSpec(memory_space=pltpu.SMEM)] + [vec_spec] * 6,
        out_specs=vec_spec,
        out_shape=jax.ShapeDtypeStruct((n_sc,), jnp.float32),
    )(tnow, infection_time, is_infected, gshape, gshift, grate, gmax)

    if n_sc == n:
        return main

    tail = pl.pallas_call(
        _tail_body,
        out_shape=jax.ShapeDtypeStruct((n - n_sc,), jnp.float32),
        in_specs=[pl.BlockSpec(memory_space=pltpu.SMEM)] + [pl.BlockSpec()] * 7,
    )(tnow, infection_time[n_sc:], infection_id[n_sc:], is_infected[n_sc:],
      param_shape[:, n_sc:], param_shift[:, n_sc:], param_rate[:, n_sc:],
      param_max[:, n_sc:])
    return jnp.concatenate([main, tail])


# speedup vs baseline: 1.3384x; 1.2053x over previous
import functools

import jax
import jax.numpy as jnp
from jax import lax
from jax.experimental import pallas as pl
from jax.experimental.pallas import tpu as pltpu
from jax.experimental.pallas import tpu_sc as plsc

_NC, _NS = 2, 16
_NW = _NC * _NS
_C = 2304

_BLOCK = 32768

_RGAMMA_COEF = (
    0.14753032712973058,
    0.37403431078058,
    1.7392577756303944,
    -1.8825767707403378,
    0.7507072601362749,
    -0.13718218631689882,
    0.007790298096042144,
    0.0009124720760137377,
    -0.00011280308480421503,
)


def _rgamma(x):
    acc = jnp.float32(_RGAMMA_COEF[-1])
    for c in reversed(_RGAMMA_COEF[:-1]):
        acc = acc * x + jnp.float32(c)
    return acc


def _sc_gather(param_shape, param_shift, param_rate, param_max, ids, n_sc):
    mesh = plsc.VectorSubcoreMesh(core_axis_name="c", subcore_axis_name="s")
    n_var = param_shape.shape[0]
    n_chunks = n_sc // _C
    out_t = [jax.ShapeDtypeStruct((n_sc,), jnp.float32)] * 4
    scratch = ([pltpu.VMEM((n_var, _C), jnp.float32)] * 2
               + [pltpu.VMEM((_C,), jnp.int32)]
               + [pltpu.VMEM((_C,), jnp.float32)] * 4
               + [pltpu.SemaphoreType.DMA] * 4)

    @functools.partial(pl.kernel, out_type=out_t, mesh=mesh,
                       scratch_types=scratch)
    def gather_k(shape_h, shift_h, rate_h, max_h, ids_h,
                 o0, o1, o2, o3, tba, tbb, idb, r0, r1, r2, r3,
                 sem_a, sem_b, sem_i, sem_o):
        wid = lax.axis_index("s") * _NC + lax.axis_index("c")
        tabs = (shape_h, shift_h, rate_h, max_h)
        tbufs = (tba, tbb)
        tsems = (sem_a, sem_b)
        rows = (r0, r1, r2, r3)
        outs = (o0, o1, o2, o3)

        def chunk(i, carry):
            g = i * _NW + wid

            @pl.when(g < n_chunks)
            def _():
                off = g * _C
                id_cp = pltpu.async_copy(ids_h.at[pl.ds(off, _C)], idb, sem_i)
                cps = [pltpu.async_copy(tabs[t].at[:, pl.ds(off, _C)],
                                        tbufs[t % 2], tsems[t % 2])
                       for t in range(2)]
                id_cp.wait()
                for t_i, r in enumerate(rows):
                    tb = tbufs[t_i % 2]
                    cps[t_i].wait()

                    def grp(k, carry2):
                        base16 = k * 16
                        ids16 = idb[pl.ds(base16, 16)]
                        acc = tb[0, pl.ds(base16, 16)]
                        for v in range(1, n_var):
                            acc = jnp.where(ids16 == v,
                                            tb[v, pl.ds(base16, 16)], acc)
                        r[pl.ds(base16, 16)] = acc
                        return carry2

                    lax.fori_loop(0, _C // 16, grp, 0)
                    if t_i + 2 < 4:
                        cps.append(pltpu.async_copy(
                            tabs[t_i + 2].at[:, pl.ds(off, _C)],
                            tbufs[t_i % 2], tsems[t_i % 2]))
                out_cps = [pltpu.async_copy(r, o.at[pl.ds(off, _C)], sem_o)
                           for o, r in zip(outs, rows)]
                for c in out_cps:
                    c.wait()

            return carry

        lax.fori_loop(0, pl.cdiv(n_chunks, _NW), chunk, 0)

    return gather_k(param_shape, param_shift, param_rate, param_max, ids)


def _compute(tnow, time, inf, shape, shift, rate, max_inf):
    t = tnow - time
    d = t - shift
    u = d * rate
    sign = jnp.where(d + 1e-10 > 0.0, 1.0, 0.0)
    val = jnp.exp((shape - 1.0) * jnp.log(u) - u) * _rgamma(shape)
    return max_inf * sign * rate * val * inf


def _math_body(tnow_ref, time_ref, inf_ref, shape_ref, shift_ref,
               rate_ref, max_ref, out_ref):
    out_ref[...] = _compute(
        tnow_ref[0], time_ref[...], inf_ref[...], shape_ref[...],
        shift_ref[...], rate_ref[...], max_ref[...])


def _tail_body(tnow_ref, time_ref, id_ref, inf_ref, shape_ref, shift_ref,
               rate_ref, max_ref, out_ref):
    ids = id_ref[...]
    n_var, blk = shape_ref.shape
    mask = jax.lax.broadcasted_iota(jnp.int32, (n_var, blk), 0) == ids[None, :]

    def sel(ref):
        return jnp.sum(jnp.where(mask, ref[...], 0.0), axis=0)

    out_ref[...] = _compute(
        tnow_ref[0], time_ref[...], inf_ref[...], sel(shape_ref),
        sel(shift_ref), sel(rate_ref), sel(max_ref))


def kernel(infection_time, infection_id, is_infected, param_shape,
           param_shift, param_rate, param_max, timer_now):
    n = infection_time.shape[0]
    n_sc = (n // _C) * _C
    tnow = jnp.asarray(timer_now, jnp.float32).reshape(1)

    gshape, gshift, grate, gmax = _sc_gather(
        param_shape, param_shift, param_rate, param_max, infection_id, n_sc)

    blk = _BLOCK
    vec_spec = pl.BlockSpec((blk,), lambda i: (i,))
    main = pl.pallas_call(
        _math_body,
        grid=(pl.cdiv(n_sc, blk),),
        in_specs=[pl.BlockSpec(memory_space=pltpu.SMEM)] + [vec_spec] * 6,
        out_specs=vec_spec,
        out_shape=jax.ShapeDtypeStruct((n_sc,), jnp.float32),
    )(tnow, infection_time, is_infected, gshape, gshift, grate, gmax)

    if n_sc == n:
        return main

    tail = pl.pallas_call(
        _tail_body,
        out_shape=jax.ShapeDtypeStruct((n - n_sc,), jnp.float32),
        in_specs=[pl.BlockSpec(memory_space=pltpu.SMEM)] + [pl.BlockSpec()] * 7,
    )(tnow, infection_time[n_sc:], infection_id[n_sc:], is_infected[n_sc:],
      param_shape[:, n_sc:], param_shift[:, n_sc:], param_rate[:, n_sc:],
      param_max[:, n_sc:])
    return jnp.concatenate([main, tail])


# speedup vs baseline: 1.3686x; 1.0226x over previous
import functools

import jax
import jax.numpy as jnp
from jax import lax
from jax.experimental import pallas as pl
from jax.experimental.pallas import tpu as pltpu
from jax.experimental.pallas import tpu_sc as plsc

_NC, _NS = 2, 16
_NW = _NC * _NS
_C = 2304

_BLOCK = 32768

_RGAMMA_COEF = (
    0.14753032712973058,
    0.37403431078058,
    1.7392577756303944,
    -1.8825767707403378,
    0.7507072601362749,
    -0.13718218631689882,
    0.007790298096042144,
    0.0009124720760137377,
    -0.00011280308480421503,
)


def _rgamma(x):
    acc = jnp.float32(_RGAMMA_COEF[-1])
    for c in reversed(_RGAMMA_COEF[:-1]):
        acc = acc * x + jnp.float32(c)
    return acc


def _sc_gather(param_shape, param_shift, param_rate, param_max, ids, n_sc):
    mesh = plsc.VectorSubcoreMesh(core_axis_name="c", subcore_axis_name="s")
    n_var = param_shape.shape[0]
    n_chunks = n_sc // _C
    out_t = [jax.ShapeDtypeStruct((n_sc,), jnp.float32)] * 4
    scratch = ([pltpu.VMEM((n_var, _C), jnp.float32)] * 2
               + [pltpu.VMEM((_C,), jnp.int32)]
               + [pltpu.VMEM((_C,), jnp.float32)] * 4
               + [pltpu.SemaphoreType.DMA] * 4)

    @functools.partial(pl.kernel, out_type=out_t, mesh=mesh,
                       scratch_types=scratch)
    def gather_k(shape_h, shift_h, rate_h, max_h, ids_h,
                 o0, o1, o2, o3, tba, tbb, idb, r0, r1, r2, r3,
                 sem_a, sem_b, sem_i, sem_o):
        wid = lax.axis_index("s") * _NC + lax.axis_index("c")
        tabs = (shape_h, shift_h, rate_h, max_h)
        tbufs = (tba, tbb)
        tsems = (sem_a, sem_b)
        rows = (r0, r1, r2, r3)
        outs = (o0, o1, o2, o3)

        def chunk(i, carry):
            g = i * _NW + wid

            @pl.when(g < n_chunks)
            def _():
                off = g * _C
                id_cp = pltpu.async_copy(ids_h.at[pl.ds(off, _C)], idb, sem_i)
                cps = [pltpu.async_copy(tabs[t].at[:, pl.ds(off, _C)],
                                        tbufs[t % 2], tsems[t % 2])
                       for t in range(2)]

                @pl.when(i > 0)
                def _drain_prev():
                    for o, r in zip(outs, rows):
                        pltpu.make_async_copy(
                            r, o.at[pl.ds(0, _C)], sem_o).wait()

                id_cp.wait()
                for t_i, r in enumerate(rows):
                    tb = tbufs[t_i % 2]
                    cps[t_i].wait()

                    def grp(k, carry2):
                        base16 = k * 16
                        ids16 = idb[pl.ds(base16, 16)]
                        acc = tb[0, pl.ds(base16, 16)]
                        for v in range(1, n_var):
                            acc = jnp.where(ids16 == v,
                                            tb[v, pl.ds(base16, 16)], acc)
                        r[pl.ds(base16, 16)] = acc
                        return carry2

                    lax.fori_loop(0, _C // 16, grp, 0)
                    if t_i + 2 < 4:
                        cps.append(pltpu.async_copy(
                            tabs[t_i + 2].at[:, pl.ds(off, _C)],
                            tbufs[t_i % 2], tsems[t_i % 2]))
                for o, r in zip(outs, rows):
                    pltpu.async_copy(r, o.at[pl.ds(off, _C)], sem_o)

            return carry

        lax.fori_loop(0, pl.cdiv(n_chunks, _NW), chunk, 0)

        @pl.when(wid < n_chunks)
        def _drain_last():
            for o, r in zip(outs, rows):
                pltpu.make_async_copy(r, o.at[pl.ds(0, _C)], sem_o).wait()

    return gather_k(param_shape, param_shift, param_rate, param_max, ids)


def _compute(tnow, time, inf, shape, shift, rate, max_inf):
    t = tnow - time
    d = t - shift
    u = d * rate
    sign = jnp.where(d + 1e-10 > 0.0, 1.0, 0.0)
    val = jnp.exp((shape - 1.0) * jnp.log(u) - u) * _rgamma(shape)
    return max_inf * sign * rate * val * inf


def _math_body(tnow_ref, time_ref, inf_ref, shape_ref, shift_ref,
               rate_ref, max_ref, out_ref):
    out_ref[...] = _compute(
        tnow_ref[0], time_ref[...], inf_ref[...], shape_ref[...],
        shift_ref[...], rate_ref[...], max_ref[...])


def _tail_body(tnow_ref, time_ref, id_ref, inf_ref, shape_ref, shift_ref,
               rate_ref, max_ref, out_ref):
    ids = id_ref[...]
    n_var, blk = shape_ref.shape
    mask = jax.lax.broadcasted_iota(jnp.int32, (n_var, blk), 0) == ids[None, :]

    def sel(ref):
        return jnp.sum(jnp.where(mask, ref[...], 0.0), axis=0)

    out_ref[...] = _compute(
        tnow_ref[0], time_ref[...], inf_ref[...], sel(shape_ref),
        sel(shift_ref), sel(rate_ref), sel(max_ref))


def kernel(infection_time, infection_id, is_infected, param_shape,
           param_shift, param_rate, param_max, timer_now):
    n = infection_time.shape[0]
    n_sc = (n // _C) * _C
    tnow = jnp.asarray(timer_now, jnp.float32).reshape(1)

    gshape, gshift, grate, gmax = _sc_gather(
        param_shape, param_shift, param_rate, param_max, infection_id, n_sc)

    blk = _BLOCK
    vec_spec = pl.BlockSpec((blk,), lambda i: (i,))
    main = pl.pallas_call(
        _math_body,
        grid=(pl.cdiv(n_sc, blk),),
        in_specs=[pl.BlockSpec(memory_space=pltpu.SMEM)] + [vec_spec] * 6,
        out_specs=vec_spec,
        out_shape=jax.ShapeDtypeStruct((n_sc,), jnp.float32),
    )(tnow, infection_time, is_infected, gshape, gshift, grate, gmax)

    if n_sc == n:
        return main

    tail = pl.pallas_call(
        _tail_body,
        out_shape=jax.ShapeDtypeStruct((n - n_sc,), jnp.float32),
        in_specs=[pl.BlockSpec(memory_space=pltpu.SMEM)] + [pl.BlockSpec()] * 7,
    )(tnow, infection_time[n_sc:], infection_id[n_sc:], is_infected[n_sc:],
      param_shape[:, n_sc:], param_shift[:, n_sc:], param_rate[:, n_sc:],
      param_max[:, n_sc:])
    return jnp.concatenate([main, tail])


# speedup vs baseline: 1.6586x; 1.2119x over previous
import functools

import jax
import jax.numpy as jnp
from jax import lax
from jax.experimental import pallas as pl
from jax.experimental.pallas import tpu as pltpu
from jax.experimental.pallas import tpu_sc as plsc

_NC, _NS = 2, 16
_NW = _NC * _NS
_C = 2304

_BLOCK = 32768

_RGAMMA_COEF = (
    0.14753032712973058,
    0.37403431078058,
    1.7392577756303944,
    -1.8825767707403378,
    0.7507072601362749,
    -0.13718218631689882,
    0.007790298096042144,
    0.0009124720760137377,
    -0.00011280308480421503,
)


def _rgamma(x):
    acc = jnp.float32(_RGAMMA_COEF[-1])
    for c in reversed(_RGAMMA_COEF[:-1]):
        acc = acc * x + jnp.float32(c)
    return acc


def _sc_gather(param_shape, param_shift, param_rate, param_max, ids, n_sc):
    mesh = plsc.VectorSubcoreMesh(core_axis_name="c", subcore_axis_name="s")
    n_var = param_shape.shape[0]
    n_chunks = n_sc // _C
    out_t = [jax.ShapeDtypeStruct((n_sc,), jnp.float32)] * 4
    scratch = ([pltpu.VMEM((n_var, _C), jnp.float32)] * 2
               + [pltpu.VMEM((_C,), jnp.int32)]
               + [pltpu.VMEM((_C,), jnp.float32)] * 4
               + [pltpu.SemaphoreType.DMA] * 4)

    @functools.partial(pl.kernel, out_type=out_t, mesh=mesh,
                       scratch_types=scratch)
    def gather_k(shape_h, shift_h, rate_h, max_h, ids_h,
                 o0, o1, o2, o3, tba, tbb, idb, r0, r1, r2, r3,
                 sem_a, sem_b, sem_i, sem_o):
        wid = lax.axis_index("s") * _NC + lax.axis_index("c")
        tabs = (shape_h, shift_h, rate_h, max_h)
        tbufs = (tba, tbb)
        tsems = (sem_a, sem_b)
        rows = (r0, r1, r2, r3)
        outs = (o0, o1, o2, o3)

        def chunk(i, carry):
            g = i * _NW + wid

            @pl.when(g < n_chunks)
            def _():
                off = g * _C

                @pl.when(i == 0)
                def _prime():
                    for t in range(2):
                        pltpu.async_copy(tabs[t].at[:, pl.ds(off, _C)],
                                         tbufs[t], tsems[t])

                id_cp = pltpu.async_copy(ids_h.at[pl.ds(off, _C)], idb, sem_i)

                @pl.when(i > 0)
                def _drain_prev():
                    for o, r in zip(outs, rows):
                        pltpu.make_async_copy(
                            r, o.at[pl.ds(0, _C)], sem_o).wait()

                id_cp.wait()
                off2 = off + _NW * _C
                for t_i, r in enumerate(rows):
                    tb = tbufs[t_i % 2]
                    pltpu.make_async_copy(
                        tabs[t_i].at[:, pl.ds(0, _C)], tb,
                        tsems[t_i % 2]).wait()

                    def grp(k, carry2):
                        base16 = k * 16
                        ids16 = idb[pl.ds(base16, 16)]
                        acc = tb[0, pl.ds(base16, 16)]
                        for v in range(1, n_var):
                            acc = jnp.where(ids16 == v,
                                            tb[v, pl.ds(base16, 16)], acc)
                        r[pl.ds(base16, 16)] = acc
                        return carry2

                    lax.fori_loop(0, _C // 16, grp, 0)
                    if t_i + 2 < 4:
                        pltpu.async_copy(
                            tabs[t_i + 2].at[:, pl.ds(off, _C)],
                            tbufs[t_i % 2], tsems[t_i % 2])
                    else:
                        @pl.when(off2 < n_chunks * _C)
                        def _prefetch():
                            pltpu.async_copy(
                                tabs[t_i - 2].at[:, pl.ds(off2, _C)],
                                tbufs[t_i % 2], tsems[t_i % 2])
                for o, r in zip(outs, rows):
                    pltpu.async_copy(r, o.at[pl.ds(off, _C)], sem_o)

            return carry

        lax.fori_loop(0, pl.cdiv(n_chunks, _NW), chunk, 0)

        @pl.when(wid < n_chunks)
        def _drain_last():
            for o, r in zip(outs, rows):
                pltpu.make_async_copy(r, o.at[pl.ds(0, _C)], sem_o).wait()

    return gather_k(param_shape, param_shift, param_rate, param_max, ids)


def _compute(tnow, time, inf, shape, shift, rate, max_inf):
    t = tnow - time
    d = t - shift
    u = d * rate
    sign = jnp.where(d + 1e-10 > 0.0, 1.0, 0.0)
    val = jnp.exp((shape - 1.0) * jnp.log(u) - u) * _rgamma(shape)
    return max_inf * sign * rate * val * inf


def _math_body(tnow_ref, time_ref, inf_ref, shape_ref, shift_ref,
               rate_ref, max_ref, out_ref):
    out_ref[...] = _compute(
        tnow_ref[0], time_ref[...], inf_ref[...], shape_ref[...],
        shift_ref[...], rate_ref[...], max_ref[...])


def _tail_body(tnow_ref, time_ref, id_ref, inf_ref, shape_ref, shift_ref,
               rate_ref, max_ref, out_ref):
    ids = id_ref[...]
    n_var, blk = shape_ref.shape
    mask = jax.lax.broadcasted_iota(jnp.int32, (n_var, blk), 0) == ids[None, :]

    def sel(ref):
        return jnp.sum(jnp.where(mask, ref[...], 0.0), axis=0)

    out_ref[...] = _compute(
        tnow_ref[0], time_ref[...], inf_ref[...], sel(shape_ref),
        sel(shift_ref), sel(rate_ref), sel(max_ref))


def kernel(infection_time, infection_id, is_infected, param_shape,
           param_shift, param_rate, param_max, timer_now):
    n = infection_time.shape[0]
    n_sc = (n // _C) * _C
    tnow = jnp.asarray(timer_now, jnp.float32).reshape(1)

    gshape, gshift, grate, gmax = _sc_gather(
        param_shape, param_shift, param_rate, param_max, infection_id, n_sc)

    blk = _BLOCK
    vec_spec = pl.BlockSpec((blk,), lambda i: (i,))
    main = pl.pallas_call(
        _math_body,
        grid=(pl.cdiv(n_sc, blk),),
        in_specs=[pl.BlockSpec(memory_space=pltpu.SMEM)] + [vec_spec] * 6,
        out_specs=vec_spec,
        out_shape=jax.ShapeDtypeStruct((n_sc,), jnp.float32),
    )(tnow, infection_time, is_infected, gshape, gshift, grate, gmax)

    if n_sc == n:
        return main

    tail = pl.pallas_call(
        _tail_body,
        out_shape=jax.ShapeDtypeStruct((n - n_sc,), jnp.float32),
        in_specs=[pl.BlockSpec(memory_space=pltpu.SMEM)] + [pl.BlockSpec()] * 7,
    )(tnow, infection_time[n_sc:], infection_id[n_sc:], is_infected[n_sc:],
      param_shape[:, n_sc:], param_shift[:, n_sc:], param_rate[:, n_sc:],
      param_max[:, n_sc:])
    return jnp.concatenate([main, tail])


# speedup vs baseline: 2.4086x; 1.4522x over previous
import functools
from math import gcd as _np_gcd

import jax
import jax.numpy as jnp
from jax import lax
from jax.experimental import pallas as pl
from jax.experimental.pallas import tpu as pltpu
from jax.experimental.pallas import tpu_sc as plsc

_NC, _NS = 2, 16
_NW = _NC * _NS
_C = 2304

_BLOCK = 32768

_RGAMMA_COEF = (
    0.14753032712973058,
    0.37403431078058,
    1.7392577756303944,
    -1.8825767707403378,
    0.7507072601362749,
    -0.13718218631689882,
    0.007790298096042144,
    0.0009124720760137377,
    -0.00011280308480421503,
)


def _rgamma(x):
    acc = jnp.float32(_RGAMMA_COEF[-1])
    for c in reversed(_RGAMMA_COEF[:-1]):
        acc = acc * x + jnp.float32(c)
    return acc


def _sc_gather(param_shape, param_shift, param_rate, param_max, ids, n_sc):
    mesh = plsc.VectorSubcoreMesh(core_axis_name="c", subcore_axis_name="s")
    n_var = param_shape.shape[0]
    n_chunks = n_sc // _C
    out_t = [jax.ShapeDtypeStruct((n_sc,), jnp.float32)] * 4
    scratch = ([pltpu.VMEM((n_var, _C), jnp.float32)] * 2
               + [pltpu.VMEM((_C,), jnp.int32)]
               + [pltpu.VMEM((_C,), jnp.float32)] * 4
               + [pltpu.SemaphoreType.DMA] * 4)

    @functools.partial(pl.kernel, out_type=out_t, mesh=mesh,
                       scratch_types=scratch)
    def gather_k(shape_h, shift_h, rate_h, max_h, ids_h,
                 o0, o1, o2, o3, tba, tbb, idb, r0, r1, r2, r3,
                 sem_a, sem_b, sem_i, sem_o):
        wid = lax.axis_index("s") * _NC + lax.axis_index("c")
        tabs = (shape_h, shift_h, rate_h, max_h)
        tbufs = (tba, tbb)
        tsems = (sem_a, sem_b)
        rows = (r0, r1, r2, r3)
        outs = (o0, o1, o2, o3)

        def chunk(i, carry):
            g = i * _NW + wid

            @pl.when(g < n_chunks)
            def _():
                off = g * _C

                @pl.when(i == 0)
                def _prime():
                    for t in range(2):
                        pltpu.async_copy(tabs[t].at[:, pl.ds(off, _C)],
                                         tbufs[t], tsems[t])

                id_cp = pltpu.async_copy(ids_h.at[pl.ds(off, _C)], idb, sem_i)

                @pl.when(i > 0)
                def _drain_prev():
                    for o, r in zip(outs, rows):
                        pltpu.make_async_copy(
                            r, o.at[pl.ds(0, _C)], sem_o).wait()

                id_cp.wait()
                off2 = off + _NW * _C
                for t_i, r in enumerate(rows):
                    tb = tbufs[t_i % 2]
                    pltpu.make_async_copy(
                        tabs[t_i].at[:, pl.ds(0, _C)], tb,
                        tsems[t_i % 2]).wait()

                    def grp(k, carry2):
                        base16 = k * 16
                        ids16 = idb[pl.ds(base16, 16)]
                        acc = tb[0, pl.ds(base16, 16)]
                        for v in range(1, n_var):
                            acc = jnp.where(ids16 == v,
                                            tb[v, pl.ds(base16, 16)], acc)
                        r[pl.ds(base16, 16)] = acc
                        return carry2

                    lax.fori_loop(0, _C // 16, grp, 0)
                    if t_i + 2 < 4:
                        pltpu.async_copy(
                            tabs[t_i + 2].at[:, pl.ds(off, _C)],
                            tbufs[t_i % 2], tsems[t_i % 2])
                    else:
                        @pl.when(off2 < n_chunks * _C)
                        def _prefetch():
                            pltpu.async_copy(
                                tabs[t_i - 2].at[:, pl.ds(off2, _C)],
                                tbufs[t_i % 2], tsems[t_i % 2])
                for o, r in zip(outs, rows):
                    pltpu.async_copy(r, o.at[pl.ds(off, _C)], sem_o)

            return carry

        lax.fori_loop(0, pl.cdiv(n_chunks, _NW), chunk, 0)

        @pl.when(wid < n_chunks)
        def _drain_last():
            for o, r in zip(outs, rows):
                pltpu.make_async_copy(r, o.at[pl.ds(0, _C)], sem_o).wait()

    return gather_k(param_shape, param_shift, param_rate, param_max, ids)


def _compute(tnow, time, inf, shape, shift, rate, max_inf):
    t = tnow - time
    d = t - shift
    u = d * rate
    sign = jnp.where(d + 1e-10 > 0.0, 1.0, 0.0)
    val = jnp.exp((shape - 1.0) * jnp.log(u) - u) * _rgamma(shape)
    return max_inf * sign * rate * val * inf


def _math_body(tnow_ref, time_ref, inf_ref, shape_ref, shift_ref,
               rate_ref, max_ref, out_ref):
    out_ref[...] = _compute(
        tnow_ref[0], time_ref[...], inf_ref[...], shape_ref[...],
        shift_ref[...], rate_ref[...], max_ref[...])


def _tail_body(tnow_ref, time_ref, id_ref, inf_ref, shape_ref, shift_ref,
               rate_ref, max_ref, out_ref):
    ids = id_ref[...]
    n_var, blk = shape_ref.shape
    mask = jax.lax.broadcasted_iota(jnp.int32, (n_var, blk), 0) == ids[None, :]

    def sel(ref):
        return jnp.sum(jnp.where(mask, ref[...], 0.0), axis=0)

    out_ref[...] = _compute(
        tnow_ref[0], time_ref[...], inf_ref[...], sel(shape_ref),
        sel(shift_ref), sel(rate_ref), sel(max_ref))


def _select_body(tnow_ref, time_ref, id_ref, inf_ref, shape_ref, shift_ref,
                 rate_ref, max_ref, out_ref):
    ids = id_ref[...]
    ids8 = (ids & 7)[None, :]
    hi_mask = ids >= 8

    def sel(ref):
        lo = jnp.take_along_axis(ref[0:8, :], ids8, axis=0)[0]
        hi = jnp.take_along_axis(ref[8:16, :], ids8, axis=0)[0]
        return jnp.where(hi_mask, hi, lo)

    out_ref[...] = _compute(
        tnow_ref[0], time_ref[...], inf_ref[...], sel(shape_ref),
        sel(shift_ref), sel(rate_ref), sel(max_ref))


def kernel(infection_time, infection_id, is_infected, param_shape,
           param_shift, param_rate, param_max, timer_now):
    n = infection_time.shape[0]
    blk = 16384
    quantum = (_C * blk) // _np_gcd(_C, blk)
    n_p = (n * 3 // 10 // quantum) * quantum
    tnow = jnp.asarray(timer_now, jnp.float32).reshape(1)

    vec_rest = pl.BlockSpec((blk,), lambda i, o=n_p // blk: (i + o,))
    tbl_rest = pl.BlockSpec((param_shape.shape[0], blk),
                            lambda i, o=n_p // blk: (0, i + o))
    out_rest = pl.BlockSpec((blk,), lambda i: (i,))
    rest = pl.pallas_call(
        _select_body,
        grid=(pl.cdiv(n - n_p, blk),),
        in_specs=[pl.BlockSpec(memory_space=pltpu.SMEM)] + [vec_rest] * 3
        + [tbl_rest] * 4,
        out_specs=out_rest,
        out_shape=jax.ShapeDtypeStruct((n - n_p,), jnp.float32),
    )(tnow, infection_time, infection_id, is_infected, param_shape,
      param_shift, param_rate, param_max)

    if n_p == 0:
        return rest

    gshape, gshift, grate, gmax = _sc_gather(
        param_shape, param_shift, param_rate, param_max, infection_id, n_p)

    vec_spec = pl.BlockSpec((blk,), lambda i: (i,))
    main = pl.pallas_call(
        _math_body,
        grid=(n_p // blk,),
        in_specs=[pl.BlockSpec(memory_space=pltpu.SMEM)] + [vec_spec] * 6,
        out_specs=vec_spec,
        out_shape=jax.ShapeDtypeStruct((n_p,), jnp.float32),
    )(tnow, infection_time, is_infected, gshape, gshift, grate, gmax)

    return jnp.concatenate([main, rest])
